# baseline (speedup 1.0000x reference)
import functools

import jax
import jax.numpy as jnp
import numpy as np
from jax.experimental import pallas as pl
from jax.experimental.pallas import tpu as pltpu

K = 2048
B = 8
ROW_BLK = 256
ALPHA = 1.1


def _sor_body(x_ref, xt_ref, sel_ref, mask_ref, val_ref, a1_ref, a2_ref):
    xb = x_ref[0]

    TB = 128
    n_tb = K // TB
    inf = jnp.float32(jnp.inf)

    pat = (jax.lax.broadcasted_iota(jnp.int32, (TB, TB), 1)
           - jax.lax.broadcasted_iota(jnp.int32, (TB, TB), 0))
    inf_tile = jnp.full((TB, TB), inf, dtype=jnp.float32)

    for blk in range(n_tb):
        a1_ref[pl.ds(blk * TB, TB), :] = inf_tile
        a2_ref[pl.ds(blk * TB, TB), :] = inf_tile

    def _insert(blk, d):
        sl = pl.ds(blk * TB, TB)
        a1 = a1_ref[sl, :]
        a2 = a2_ref[sl, :]
        a2_ref[sl, :] = jnp.minimum(a2, jnp.maximum(a1, d))
        a1_ref[sl, :] = jnp.minimum(a1, d)

    for s in range(n_tb):
        for bi in range(n_tb - s):
            bj = bi + s
            r0 = x_ref[0, pl.ds(bi * TB, TB), 0:1]
            r1 = x_ref[0, pl.ds(bi * TB, TB), 1:2]
            r2 = x_ref[0, pl.ds(bi * TB, TB), 2:3]
            c0 = xt_ref[0, 0:1, bj * TB:(bj + 1) * TB]
            c1 = xt_ref[0, 1:2, bj * TB:(bj + 1) * TB]
            c2 = xt_ref[0, 2:3, bj * TB:(bj + 1) * TB]
            e0 = r0 - c0
            e1 = r1 - c1
            e2 = r2 - c2
            d = e0 * e0 + e1 * e1 + e2 * e2
            if bi == bj:
                d = jnp.where(pat == 0, inf, d)
            _insert(bi, d)
            if bj > bi:
                _insert(bj, d.T)

    for blk in range(n_tb):
        sl = pl.ds(blk * TB, TB)
        a1 = a1_ref[sl, :]
        a2 = a2_ref[sl, :]
        g1 = jnp.min(a1, axis=1, keepdims=True)
        zm = a1 == g1
        cnt = jnp.sum(zm.astype(jnp.float32), axis=1, keepdims=True)
        s2 = jnp.min(jnp.where(zm, inf, a1), axis=1, keepdims=True)
        m2a = jnp.min(a2, axis=1, keepdims=True)
        d3 = jnp.where(cnt >= 2.0, g1, jnp.minimum(s2, m2a))
        val_ref[sl, :] = 0.5 * (g1 + d3)

    v = val_ref[:, :]
    mean = jnp.sum(v) * (1.0 / K)
    centered = v - mean
    var = jnp.sum(centered * centered) * (1.0 / (K - 1))
    threshold = mean + ALPHA * jnp.sqrt(var)
    keep = (v <= threshold).astype(jnp.float32)

    sel_ref[0] = xb * keep
    mask_ref[0] = keep


def _sor_pallas(x, xt):
    b = x.shape[0]
    _imap = lambda i: (i, i * 0, i * 0)
    return pl.pallas_call(
        _sor_body,
        grid=(b,),
        in_specs=[
            pl.BlockSpec((1, K, 3), _imap),
            pl.BlockSpec((1, 3, K), _imap),
        ],
        out_specs=[
            pl.BlockSpec((1, K, 3), _imap),
            pl.BlockSpec((1, K, 1), _imap),
        ],
        out_shape=[
            jax.ShapeDtypeStruct((b, K, 3), jnp.float32),
            jax.ShapeDtypeStruct((b, K, 1), jnp.float32),
        ],
        scratch_shapes=[
            pltpu.VMEM((K, 1), jnp.float32),
            pltpu.VMEM((K, 128), jnp.float32),
            pltpu.VMEM((K, 128), jnp.float32),
        ],
        compiler_params=pltpu.CompilerParams(
            dimension_semantics=("parallel",),
        ),
    )(x, xt)


@jax.jit
def kernel(x):
    xt = jnp.swapaxes(x, 1, 2)
    sel, maskf = _sor_pallas(x, xt)
    mask = maskf[:, :, 0] > 0.5
    return sel, mask


# speedup vs baseline: 1.5731x; 1.5731x over previous
import functools

import jax
import jax.numpy as jnp
import numpy as np
from jax.experimental import pallas as pl
from jax.experimental.pallas import tpu as pltpu

K = 2048
B = 8
ROW_BLK = 256
ALPHA = 1.1


def _sor_body(x_ref, xt_ref, sel_ref, mask_ref, val_ref, a1_ref, a2_ref):
    xb = x_ref[0]

    TB = 128
    n_tb = K // TB
    inf = jnp.float32(jnp.inf)

    pat = (jax.lax.broadcasted_iota(jnp.int32, (TB, TB), 1)
           - jax.lax.broadcasted_iota(jnp.int32, (TB, TB), 0))
    inf_tile = jnp.full((TB, TB), inf, dtype=jnp.float32)

    for blk in range(n_tb):
        a1_ref[pl.ds(blk * TB, TB), :] = inf_tile
        a2_ref[pl.ds(blk * TB, TB), :] = inf_tile

    def _insert(blk, d):
        sl = pl.ds(blk * TB, TB)
        a1 = a1_ref[sl, :]
        a2 = a2_ref[sl, :]
        a2_ref[sl, :] = jnp.minimum(a2, jnp.maximum(a1, d))
        a1_ref[sl, :] = jnp.minimum(a1, d)

    for bi in range(n_tb):
        r0 = x_ref[0, pl.ds(bi * TB, TB), 0:1]
        r1 = x_ref[0, pl.ds(bi * TB, TB), 1:2]
        r2 = x_ref[0, pl.ds(bi * TB, TB), 2:3]
        for bj in range(bi, n_tb):
            c0 = xt_ref[0, 0:1, bj * TB:(bj + 1) * TB]
            c1 = xt_ref[0, 1:2, bj * TB:(bj + 1) * TB]
            c2 = xt_ref[0, 2:3, bj * TB:(bj + 1) * TB]
            e0 = r0 - c0
            e1 = r1 - c1
            e2 = r2 - c2
            d = e0 * e0 + e1 * e1 + e2 * e2
            if bi == bj:
                d = jnp.where(pat == 0, inf, d)
            _insert(bi, d)
            if bj > bi:
                _insert(bj, d.T)

    for blk in range(n_tb):
        sl = pl.ds(blk * TB, TB)
        a1 = a1_ref[sl, :]
        a2 = a2_ref[sl, :]
        g1 = jnp.min(a1, axis=1, keepdims=True)
        zm = a1 == g1
        cnt = jnp.sum(zm.astype(jnp.float32), axis=1, keepdims=True)
        s2 = jnp.min(jnp.where(zm, inf, a1), axis=1, keepdims=True)
        m2a = jnp.min(a2, axis=1, keepdims=True)
        d3 = jnp.where(cnt >= 2.0, g1, jnp.minimum(s2, m2a))
        val_ref[sl, :] = 0.5 * (g1 + d3)

    v = val_ref[:, :]
    mean = jnp.sum(v) * (1.0 / K)
    centered = v - mean
    var = jnp.sum(centered * centered) * (1.0 / (K - 1))
    threshold = mean + ALPHA * jnp.sqrt(var)
    keep = (v <= threshold).astype(jnp.float32)

    sel_ref[0] = xb * keep
    mask_ref[0] = keep


def _sor_pallas(x, xt):
    b = x.shape[0]
    _imap = lambda i: (i, i * 0, i * 0)
    return pl.pallas_call(
        _sor_body,
        grid=(b,),
        in_specs=[
            pl.BlockSpec((1, K, 3), _imap),
            pl.BlockSpec((1, 3, K), _imap),
        ],
        out_specs=[
            pl.BlockSpec((1, K, 3), _imap),
            pl.BlockSpec((1, K, 1), _imap),
        ],
        out_shape=[
            jax.ShapeDtypeStruct((b, K, 3), jnp.float32),
            jax.ShapeDtypeStruct((b, K, 1), jnp.float32),
        ],
        scratch_shapes=[
            pltpu.VMEM((K, 1), jnp.float32),
            pltpu.VMEM((K, 128), jnp.float32),
            pltpu.VMEM((K, 128), jnp.float32),
        ],
        compiler_params=pltpu.CompilerParams(
            dimension_semantics=("parallel",),
        ),
    )(x, xt)


@jax.jit
def kernel(x):
    xt = jnp.swapaxes(x, 1, 2)
    sel, maskf = _sor_pallas(x, xt)
    mask = maskf[:, :, 0] > 0.5
    return sel, mask
